# baseline (speedup 1.0000x reference)
import functools

import jax
import jax.numpy as jnp
from jax import lax
from jax.experimental import pallas as pl
from jax.experimental.pallas import tpu as pltpu
from jax.experimental.pallas import tpu_sc as plsc

_B, _K, _C, _T = 256, 1024, 32, 8
_D = _C * _T

_HI = jax.lax.Precision.HIGHEST

_NC = 2
_NS = 16
_NW = _NC * _NS
_KPW = _K // _NW


def _vq_body(yt_ref, mt_ref, sd_ref, z_ref, mo_ref, sdo_ref):
    yt = yt_ref[:]
    mt = mt_ref[:]

    g = jax.lax.dot_general(yt, mt, (((0,), (0,)), ((), ())),
                            precision=_HI)
    mm = mt * mt
    h = _D
    while h > 1:
        h //= 2
        mm = mm[:h, :] + mm[h:, :]
    d2 = mm - 2.0 * g

    kiota = jax.lax.broadcasted_iota(jnp.int32, (_B, _K), 1)
    biota = jax.lax.broadcasted_iota(jnp.int32, (_B, _K), 0)

    dmin = jnp.min(d2, axis=1, keepdims=True)
    z2 = jnp.min(jnp.where(d2 == dmin, kiota, _K), axis=1,
                 keepdims=True)
    z_ref[:] = z2

    onehot = z2 == kiota
    iwin = jnp.max(jnp.where(onehot, biota, -1), axis=0,
                   keepdims=True)
    win = ((biota == iwin) & (iwin >= 0)).astype(jnp.float32)
    ywt = jax.lax.dot_general(yt, win, (((1,), (0,)), ((), ())),
                              precision=_HI)
    assigned = iwin >= 0

    mn = mt * 0.01 + ywt * 0.99
    mo_ref[:] = jnp.where(assigned, mn, mt)
    dlt = mn - ywt
    sdt = sd_ref[:]
    sdo_ref[:] = jnp.where(assigned, dlt * dlt * 0.01 + sdt * 0.99, sdt)


@functools.partial(
    pl.kernel,
    mesh=plsc.VectorSubcoreMesh(core_axis_name="c", subcore_axis_name="s"),
    out_type=jax.ShapeDtypeStruct((_K,), jnp.float32),
    scratch_types=[
        pltpu.VMEM((_B,), jnp.int32),
        pltpu.VMEM((_KPW,), jnp.float32),
        pltpu.VMEM((_KPW,), jnp.float32),
    ],
)
def _p_count(z_hbm, p_hbm, out_hbm, z_v, p_v, out_v):
    wid = lax.axis_index("s") * _NC + lax.axis_index("c")
    base = wid * _KPW
    pltpu.sync_copy(z_hbm, z_v)
    pltpu.sync_copy(p_hbm.at[pl.ds(base, _KPW)], p_v)
    lane = lax.broadcasted_iota(jnp.int32, (16,), 0)
    ids0 = lane + base
    ids1 = lane + (base + 16)
    zero = jnp.zeros((16,), jnp.float32)
    one = jnp.ones((16,), jnp.float32)

    def chunk_body(c, carry):
        a0, a1 = carry
        zc = z_v[pl.ds(c * 16, 16)]
        for r in range(16):
            idx = (lane + r) & 15
            zr = zc.at[idx].get(mode="promise_in_bounds")
            a0 = a0 + jnp.where(ids0 == zr, one, zero)
            a1 = a1 + jnp.where(ids1 == zr, one, zero)
        return (a0, a1)

    a0, a1 = lax.fori_loop(0, _B // 16, chunk_body, (zero, zero))
    out_v[pl.ds(0, 16)] = p_v[pl.ds(0, 16)] + a0
    out_v[pl.ds(16, 16)] = p_v[pl.ds(16, 16)] + a1
    pltpu.sync_copy(out_v, out_hbm.at[pl.ds(base, _KPW)])


def kernel(y, m, sd, p):
    yt = y.reshape(_B, _D).T
    mt = m.reshape(_K, _D).T
    sdt = sd.reshape(_K, _D).T
    z2, mo, sdo = pl.pallas_call(
        _vq_body,
        out_shape=(
            jax.ShapeDtypeStruct((_B, 1), jnp.int32),
            jax.ShapeDtypeStruct((_D, _K), jnp.float32),
            jax.ShapeDtypeStruct((_D, _K), jnp.float32),
        ),
    )(yt, mt, sdt)
    z1 = z2.reshape(_B)
    po = _p_count(z1, p)
    return (z1, mo.T.reshape(_K, _C, _T), sdo.T.reshape(_K, _C, _T), po)


# speedup vs baseline: 3.2333x; 3.2333x over previous
import jax
import jax.numpy as jnp
from jax.experimental import pallas as pl

_B, _K, _C, _T = 256, 1024, 32, 8
_D = _C * _T

_HI = jax.lax.Precision.HIGHEST


def _vq_body(yt_ref, mt_ref, sd_ref, p_ref, z_ref, mo_ref, sdo_ref, po_ref):
    yt = yt_ref[:]
    mt = mt_ref[:]

    g = jax.lax.dot_general(yt, mt, (((0,), (0,)), ((), ())),
                            precision=_HI)
    mm = mt * mt
    h = _D
    while h > 1:
        h //= 2
        mm = mm[:h, :] + mm[h:, :]
    d2 = mm - 2.0 * g

    kiota = jax.lax.broadcasted_iota(jnp.int32, (_B, _K), 1)
    biota = jax.lax.broadcasted_iota(jnp.int32, (_B, _K), 0)

    dmin = jnp.min(d2, axis=1, keepdims=True)
    z2 = jnp.min(jnp.where(d2 == dmin, kiota, _K), axis=1,
                 keepdims=True)
    z_ref[:] = z2

    onehot = z2 == kiota
    iwin = jnp.max(jnp.where(onehot, biota, -1), axis=0,
                   keepdims=True)
    count = jnp.sum(onehot.astype(jnp.float32), axis=0,
                    keepdims=True)
    po_ref[:] = p_ref[:] + count

    win = ((biota == iwin) & (iwin >= 0)).astype(jnp.float32)
    ywt = jax.lax.dot_general(yt, win, (((1,), (0,)), ((), ())),
                              precision=_HI)
    assigned = iwin >= 0

    mn = mt * 0.01 + ywt * 0.99
    mo_ref[:] = jnp.where(assigned, mn, mt)
    dlt = mn - ywt
    sdt = sd_ref[:]
    sdo_ref[:] = jnp.where(assigned, dlt * dlt * 0.01 + sdt * 0.99, sdt)


def kernel(y, m, sd, p):
    yt = y.reshape(_B, _D).T
    mt = m.reshape(_K, _D).T
    sdt = sd.reshape(_K, _D).T
    p2 = p.reshape(1, _K)
    z2, mo, sdo, po = pl.pallas_call(
        _vq_body,
        out_shape=(
            jax.ShapeDtypeStruct((_B, 1), jnp.int32),
            jax.ShapeDtypeStruct((_D, _K), jnp.float32),
            jax.ShapeDtypeStruct((_D, _K), jnp.float32),
            jax.ShapeDtypeStruct((1, _K), jnp.float32),
        ),
    )(yt, mt, sdt, p2)
    return (z2.reshape(_B), mo.T.reshape(_K, _C, _T),
            sdo.T.reshape(_K, _C, _T), po.reshape(_K))
